# baseline (speedup 1.0000x reference)
import numpy as np
import jax
import jax.numpy as jnp
from jax.experimental import pallas as pl
from jax.experimental.pallas import tpu as pltpu

_K = 25
_PAD = 12


def _band_weight(l: int) -> np.ndarray:
    w = np.zeros((l, l), np.float32)
    for t in range(l):
        for j in range(_K):
            src = min(max(t + j - _PAD, 0), l - 1)
            w[t, src] += 1.0 / _K
    return w


def _ma_body(w_ref, x_ref, o_ref):
    w = w_ref[...]
    for b in range(x_ref.shape[0]):
        xb = x_ref[b].astype(jnp.bfloat16)
        o_ref[b] = jnp.dot(w, xb, preferred_element_type=jnp.float32)


def kernel(x):
    b, l, c = x.shape
    w = jnp.asarray(_band_weight(l), dtype=jnp.bfloat16)
    b_blk = 1
    for cand in (4, 8, 2):
        if b % cand == 0 and b // cand >= 2:
            b_blk = cand
            break
    block_bytes = b_blk * l * c * x.dtype.itemsize
    vmem_limit = int(min(max(6 * block_bytes, 16 << 20), 64 << 20))
    return pl.pallas_call(
        _ma_body,
        out_shape=jax.ShapeDtypeStruct((b, l, c), x.dtype),
        grid=(b // b_blk,),
        in_specs=[
            pl.BlockSpec((l, l), lambda i: (0, 0)),
            pl.BlockSpec((b_blk, l, c), lambda i: (i, 0, 0)),
        ],
        out_specs=pl.BlockSpec((b_blk, l, c), lambda i: (i, 0, 0)),
        compiler_params=pltpu.CompilerParams(
            dimension_semantics=("parallel",),
            vmem_limit_bytes=vmem_limit,
        ),
    )(w, x)


# speedup vs baseline: 1.0464x; 1.0464x over previous
import numpy as np
import jax
import jax.numpy as jnp
from jax.experimental import pallas as pl
from jax.experimental.pallas import tpu as pltpu

_K = 25
_PAD = 12
_KT = 256
_M_CHUNK = 224


def _chunk_plan(l):
    chunks = []
    s = 0
    while s < l:
        m = min(_M_CHUNK, l - s)
        lo = max(0, s - _PAD)
        hi = min(l, s + m + _PAD)
        a_min = max(0, hi - _KT)
        a = -((-a_min) // 8) * 8
        assert a <= min(lo, l - _KT), (s, m, a)
        chunks.append((s, m, a))
        s += m
    return chunks


def _band_weights(l, chunks):
    w = np.zeros((l, _KT), np.float32)
    for s, m, a in chunks:
        for t in range(s, s + m):
            for j in range(_K):
                src = min(max(t + j - _PAD, 0), l - 1)
                w[t, src - a] += 1.0 / _K
    return w


def _ma_body(chunks, w_ref, x_ref, o_ref):
    w = w_ref[...]
    for b in range(x_ref.shape[0]):
        xb = x_ref[b].astype(jnp.bfloat16)
        for s, m, a in chunks:
            o_ref[b, s : s + m, :] = jnp.dot(
                w[s : s + m, :],
                xb[a : a + _KT, :],
                preferred_element_type=jnp.float32,
            )


def kernel(x):
    import functools

    b, l, c = x.shape
    chunks = _chunk_plan(l)
    w = jnp.asarray(_band_weights(l, chunks), dtype=jnp.bfloat16)
    b_blk = 1
    for cand in (8, 4, 2):
        if b % cand == 0 and b // cand >= 2:
            b_blk = cand
            break
    block_bytes = b_blk * l * c * x.dtype.itemsize
    vmem_limit = int(min(max(6 * block_bytes, 16 << 20), 64 << 20))
    return pl.pallas_call(
        functools.partial(_ma_body, chunks),
        out_shape=jax.ShapeDtypeStruct((b, l, c), x.dtype),
        grid=(b // b_blk,),
        in_specs=[
            pl.BlockSpec((l, _KT), lambda i: (0, 0)),
            pl.BlockSpec((b_blk, l, c), lambda i: (i, 0, 0)),
        ],
        out_specs=pl.BlockSpec((b_blk, l, c), lambda i: (i, 0, 0)),
        compiler_params=pltpu.CompilerParams(
            dimension_semantics=("parallel",),
            vmem_limit_bytes=vmem_limit,
        ),
    )(w, x)


# speedup vs baseline: 1.0474x; 1.0010x over previous
import functools

import numpy as np
import jax
import jax.numpy as jnp
from jax.experimental import pallas as pl
from jax.experimental.pallas import tpu as pltpu

_K = 25
_PAD = 12
_KT = 256
_M_CHUNK = 224


def _chunk_plan(l):
    chunks = []
    s = 0
    while s < l:
        m = min(_M_CHUNK, l - s)
        lo = max(0, s - _PAD)
        hi = min(l, s + m + _PAD)
        a_min = max(0, hi - _KT)
        a = -((-a_min) // 8) * 8
        assert a <= min(lo, l - _KT), (s, m, a)
        chunks.append((s, m, a))
        s += m
    return chunks


def _band_weights(l, chunks):
    w = np.zeros((l, _KT), np.float32)
    for s, m, a in chunks:
        for t in range(s, s + m):
            for j in range(_K):
                src = min(max(t + j - _PAD, 0), l - 1)
                w[t, src - a] += 1.0 / _K
    return w


def _ma_body(chunks, w_ref, x_ref, o_ref):
    w = w_ref[...]
    for b in range(x_ref.shape[0]):
        xb = x_ref[b].astype(jnp.bfloat16)
        for s, m, a in chunks:
            o_ref[b, s : s + m, :] = jnp.dot(
                w[s : s + m, :],
                xb[a : a + _KT, :],
                preferred_element_type=jnp.float32,
            )


def kernel(x):
    b, l, c = x.shape
    chunks = _chunk_plan(l)
    w = jnp.asarray(_band_weights(l, chunks), dtype=jnp.bfloat16)
    b_blk = 1
    for cand in (8, 4, 2):
        if b % cand == 0 and b // cand >= 2:
            b_blk = cand
            break
    block_bytes = b_blk * l * c * x.dtype.itemsize
    vmem_limit = int(min(max(6 * block_bytes, 16 << 20), 64 << 20))
    return pl.pallas_call(
        functools.partial(_ma_body, chunks),
        out_shape=jax.ShapeDtypeStruct((b, l, c), x.dtype),
        grid=(b // b_blk,),
        in_specs=[
            pl.BlockSpec((l, _KT), lambda i: (0, 0)),
            pl.BlockSpec((b_blk, l, c), lambda i: (i, 0, 0)),
        ],
        out_specs=pl.BlockSpec((b_blk, l, c), lambda i: (i, 0, 0)),
        compiler_params=pltpu.CompilerParams(
            dimension_semantics=("parallel",),
            vmem_limit_bytes=vmem_limit,
        ),
    )(w, x)
